# baseline (speedup 1.0000x reference)
import functools
import math

import jax
import jax.numpy as jnp
from jax import lax
from jax.experimental import pallas as pl
from jax.experimental.pallas import tpu as pltpu
from jax.experimental.pallas import tpu_sc as plsc

NUM_EMB = 100000
EMB_DIM = 64
BATCH = 16384

NC = 2
NS = 16
NW = NC * NS
B_PER_W = BATCH // NW
CHUNK = 128
NCHUNK = B_PER_W // CHUNK


def _sc_gather(table, ids2d):
    mesh = plsc.VectorSubcoreMesh(core_axis_name="c", subcore_axis_name="s")

    @functools.partial(
        pl.kernel,
        out_type=jax.ShapeDtypeStruct((BATCH, EMB_DIM), jnp.float32),
        mesh=mesh,
        scratch_types=[
            pltpu.VMEM((NCHUNK, CHUNK), jnp.int32),
            pltpu.VMEM((B_PER_W, EMB_DIM), jnp.float32),
            pltpu.SemaphoreType.DMA,
        ],
        compiler_params=pltpu.CompilerParams(use_tc_tiling_on_sc=False),
    )
    def k(table_hbm, idx_hbm, out_hbm, idx_v, rows_v, sem):
        wid = lax.axis_index("s") * NC + lax.axis_index("c")
        base = wid * B_PER_W
        pltpu.sync_copy(idx_hbm.at[pl.ds(wid * NCHUNK, NCHUNK)], idx_v)
        copies = []
        for j in range(NCHUNK):
            copies.append(
                pltpu.async_copy(
                    table_hbm.at[idx_v.at[j]],
                    rows_v.at[pl.ds(j * CHUNK, CHUNK)],
                    sem,
                )
            )
        for c in copies:
            c.wait()
        pltpu.sync_copy(rows_v, out_hbm.at[pl.ds(base, B_PER_W)])

    return k(table, ids2d)


_ROWS_BLK = 2048
_INV_SQRT2 = 1.0 / math.sqrt(2.0)


def _tc_body(emb_ref, wt_ref, b_ref, out_ref):
    proj = jnp.dot(emb_ref[...], wt_ref[...],
                   preferred_element_type=jnp.float32) + b_ref[...]
    out_ref[...] = 0.5 * proj * (1.0 + lax.erf(proj * _INV_SQRT2))


def _tc_project_gelu(emb, w_t, b2d):
    grid = BATCH // _ROWS_BLK
    return pl.pallas_call(
        _tc_body,
        grid=(grid,),
        in_specs=[
            pl.BlockSpec((_ROWS_BLK, EMB_DIM), lambda i: (i, 0)),
            pl.BlockSpec((EMB_DIM, EMB_DIM), lambda i: (0, 0)),
            pl.BlockSpec((1, EMB_DIM), lambda i: (0, 0)),
        ],
        out_specs=pl.BlockSpec((_ROWS_BLK, EMB_DIM), lambda i: (i, 0)),
        out_shape=jax.ShapeDtypeStruct((BATCH, EMB_DIM), jnp.float32),
    )(emb, w_t, b2d)


def kernel(country_ids, table, W, b):
    ids2d = country_ids.astype(jnp.int32).reshape(BATCH // CHUNK, CHUNK)
    emb = _sc_gather(table, ids2d)
    return _tc_project_gelu(emb, W.T, b.reshape(1, EMB_DIM))


# speedup vs baseline: 1.0855x; 1.0855x over previous
import functools
import math

import jax
import jax.numpy as jnp
from jax import lax
from jax.experimental import pallas as pl
from jax.experimental.pallas import tpu as pltpu
from jax.experimental.pallas import tpu_sc as plsc

NUM_EMB = 100000
EMB_DIM = 64
BATCH = 16384

NC = 2
NS = 16
NW = NC * NS
B_PER_W = BATCH // NW
CHUNK = 128
NCHUNK = B_PER_W // CHUNK


def _sc_gather(table, ids):
    mesh = plsc.VectorSubcoreMesh(core_axis_name="c", subcore_axis_name="s")

    @functools.partial(
        pl.kernel,
        out_type=jax.ShapeDtypeStruct((BATCH, EMB_DIM), jnp.float32),
        mesh=mesh,
        scratch_types=[
            pltpu.VMEM((B_PER_W,), jnp.int32),
            pltpu.VMEM((B_PER_W, EMB_DIM), jnp.float32),
            pltpu.SemaphoreType.DMA,
        ],
        compiler_params=pltpu.CompilerParams(use_tc_tiling_on_sc=False),
    )
    def k(table_hbm, idx_hbm, out_hbm, idx_v, rows_v, sem):
        wid = lax.axis_index("s") * NC + lax.axis_index("c")
        base = wid * B_PER_W
        pltpu.sync_copy(idx_hbm.at[pl.ds(base, B_PER_W)], idx_v)
        copies = []
        for j in range(NCHUNK):
            copies.append(
                pltpu.async_copy(
                    table_hbm.at[idx_v.at[pl.ds(j * CHUNK, CHUNK)]],
                    rows_v.at[pl.ds(j * CHUNK, CHUNK)],
                    sem,
                )
            )
        for c in copies:
            c.wait()
        pltpu.sync_copy(rows_v, out_hbm.at[pl.ds(base, B_PER_W)])

    return k(table, ids)


_PAIR_BLK = 1024
_INV_SQRT2 = 1.0 / math.sqrt(2.0)


def _tc_body(emb2_ref, wbig_ref, bbig_ref, out_ref):
    proj = jnp.dot(emb2_ref[...], wbig_ref[...],
                   preferred_element_type=jnp.float32) + bbig_ref[...]
    act = 0.5 * proj * (1.0 + lax.erf(proj * _INV_SQRT2))
    out_ref[pl.Slice(0, _PAIR_BLK, 2), :] = act[:, :EMB_DIM]
    out_ref[pl.Slice(1, _PAIR_BLK, 2), :] = act[:, EMB_DIM:]


def _tc_project_gelu(emb2, w_big, b_big):
    grid = (BATCH // 2) // _PAIR_BLK
    return pl.pallas_call(
        _tc_body,
        grid=(grid,),
        in_specs=[
            pl.BlockSpec((_PAIR_BLK, 2 * EMB_DIM), lambda i: (i, 0)),
            pl.BlockSpec((2 * EMB_DIM, 2 * EMB_DIM), lambda i: (0, 0)),
            pl.BlockSpec((1, 2 * EMB_DIM), lambda i: (0, 0)),
        ],
        out_specs=pl.BlockSpec((2 * _PAIR_BLK, EMB_DIM), lambda i: (i, 0)),
        out_shape=jax.ShapeDtypeStruct((BATCH, EMB_DIM), jnp.float32),
    )(emb2, w_big, b_big)


def kernel(country_ids, table, W, b):
    ids = country_ids.astype(jnp.int32)
    emb = _sc_gather(table, ids)
    emb2 = emb.reshape(BATCH // 2, 2 * EMB_DIM)
    wt = W.T
    zero = jnp.zeros((EMB_DIM, EMB_DIM), jnp.float32)
    w_big = jnp.block([[wt, zero], [zero, wt]])
    b_big = jnp.concatenate([b, b]).reshape(1, 2 * EMB_DIM)
    return _tc_project_gelu(emb2, w_big, b_big)
